# Initial kernel scaffold; baseline (speedup 1.0000x reference)
import jax
import jax.numpy as jnp
from jax.experimental import pallas as pl


def kernel(input):
    raise NotImplementedError("write your pallas kernel here")


# speedup vs baseline: 1.5892x; 1.5892x over previous
import jax
import jax.numpy as jnp
import numpy as np
from jax.experimental import pallas as pl

_BS = 16
_A = 3
_C = 80
_ATTRS = 5 + _C
_H = 76
_W = 76
_HW = _H * _W
_STRIDE = 8.0
_ANCHORS = np.array([[116.0, 90.0], [156.0, 198.0], [373.0, 326.0]],
                    dtype=np.float32)
_SCALED = _ANCHORS / _STRIDE


def _decode_kernel(x_ref, o_ref):
    a = pl.program_id(1)
    x = x_ref[0, 0]
    s = jax.nn.sigmoid(x)

    col = jax.lax.broadcasted_iota(jnp.int32, (1, _HW), 1)
    gx = (col % _W).astype(jnp.float32)
    gy = (col // _W).astype(jnp.float32)

    row0 = (s[0:1, :] + gx) * _STRIDE
    row1 = (s[1:2, :] + gy) * _STRIDE

    aw = jnp.where(a == 0, _ANCHORS[0, 0],
                   jnp.where(a == 1, _ANCHORS[1, 0], _ANCHORS[2, 0]))
    ah = jnp.where(a == 0, _ANCHORS[0, 1],
                   jnp.where(a == 1, _ANCHORS[1, 1], _ANCHORS[2, 1]))
    row2 = jnp.exp(x[2:3, :]) * aw
    row3 = jnp.exp(x[3:4, :]) * ah

    r = jnp.concatenate([row0, row1, row2, row3, s[4:, :]], axis=0)
    o_ref[0, 0] = r.T


def kernel(input):
    x = input.reshape(_BS, _A, _ATTRS, _HW)
    out = pl.pallas_call(
        _decode_kernel,
        grid=(_BS, _A),
        in_specs=[
            pl.BlockSpec((1, 1, _ATTRS, _HW), lambda b, a: (b, a, 0, 0)),
        ],
        out_specs=pl.BlockSpec((1, 1, _HW, _ATTRS), lambda b, a: (b, a, 0, 0)),
        out_shape=jax.ShapeDtypeStruct((_BS, _A, _HW, _ATTRS), jnp.float32),
    )(x)
    return out.reshape(_BS, _A * _HW, _ATTRS)
